# Initial kernel scaffold; baseline (speedup 1.0000x reference)
import jax
import jax.numpy as jnp
from jax.experimental import pallas as pl


def kernel(x, edge_index, edge_attr, We1, be1, g1, b1, m1, v1, We2, be2, g2, b2, m2, v2, Wm, bm, Wu, bu, Wd1, bd1, Wd2, bd2):
    raise NotImplementedError("write your pallas kernel here")


# speedup vs baseline: 2.6000x; 2.6000x over previous
import functools

import jax
import jax.numpy as jnp
from jax import lax
from jax.experimental import pallas as pl
from jax.experimental.pallas import tpu as pltpu
from jax.experimental.pallas import tpu_sc as plsc

N = 10000
E = 640000
H = 128
HH = 64
NSUB = 16
NCORE = 2
EPS = E // NSUB
G = 5
CE = G * 64
NCHUNK = EPS // CE
NPS = N // NSUB


def _edge_kernel(pflat, src1, dst1, attr1, wme1):
    mesh = plsc.VectorSubcoreMesh(core_axis_name="c", subcore_axis_name="s")

    @functools.partial(
        pl.kernel,
        out_type=jax.ShapeDtypeStruct((NCORE * N, HH), jnp.float32),
        mesh=mesh,
        compiler_params=pltpu.CompilerParams(use_tc_tiling_on_sc=False),
        scratch_types=(
            [pltpu.VMEM((64,), jnp.int32) for _ in range(G)]
            + [pltpu.VMEM((64,), jnp.int32) for _ in range(G)]
            + [
                pltpu.VMEM((3 * CE,), jnp.float32),
                pltpu.VMEM((CE, HH), jnp.float32),
                pltpu.VMEM((3 * HH,), jnp.float32),
                pltpu.VMEM_SHARED((N, HH), jnp.float32),
                pltpu.SemaphoreType.DMA,
            ]
        ),
    )
    def body(p_ref, src_ref, dst_ref, attr_ref, wme_ref, agg_ref, *scr):
        idxs_g = scr[:G]
        idxd_g = scr[G:2 * G]
        attr_v, rows_v, wm_v, agg_sh, sem = scr[2 * G:]
        c = lax.axis_index("c")
        s = lax.axis_index("s")

        pltpu.sync_copy(wme_ref.at[pl.ds(c * 3 * HH, 3 * HH)], wm_v)

        zero = jnp.zeros((16,), jnp.float32)

        def zrow(i, carry):
            for k in range(HH // 16):
                rows_v[i, pl.ds(k * 16, 16)] = zero
            return carry

        lax.fori_loop(0, CE, zrow, 0)
        n0 = s * 624
        pltpu.sync_copy(rows_v, agg_sh.at[pl.ds(n0, CE)])
        pltpu.sync_copy(rows_v.at[pl.ds(0, 624 - CE)],
                        agg_sh.at[pl.ds(n0 + CE, 624 - CE)])

        @pl.when(s == 0)
        def _():
            pltpu.sync_copy(rows_v.at[pl.ds(0, 16)],
                            agg_sh.at[pl.ds(N - 16, 16)])

        plsc.subcore_barrier()

        w = [[wm_v[pl.ds(j * HH + k * 16, 16)] for k in range(HH // 16)]
             for j in range(3)]
        coff = c * N

        def chunk(t, carry):
            e0 = s * EPS + t * CE
            for g in range(G):
                pltpu.sync_copy(src_ref.at[pl.ds(e0 + g * 64, 64)],
                                idxs_g[g])
                pltpu.sync_copy(dst_ref.at[pl.ds(e0 + g * 64, 64)],
                                idxd_g[g])
            for j in range(3):
                pltpu.sync_copy(attr_ref.at[pl.ds(j * E + e0, CE)],
                                attr_v.at[pl.ds(j * CE, CE)])
            for g in range(G):
                for k in range(4):
                    idxs_g[g][pl.ds(k * 16, 16)] = (
                        idxs_g[g][pl.ds(k * 16, 16)] + coff)
            descs = [
                pltpu.async_copy(p_ref.at[idxs_g[g]],
                                 rows_v.at[pl.ds(g * 64, 64)], sem)
                for g in range(G)
            ]
            for d in descs:
                d.wait()

            def grp(q, carry2):
                i0 = q * 16
                a0g = attr_v[pl.ds(i0, 16)]
                a1g = attr_v[pl.ds(CE + i0, 16)]
                a2g = attr_v[pl.ds(2 * CE + i0, 16)]
                for j in range(16):
                    i = i0 + j
                    a0 = a0g[j]
                    a1 = a1g[j]
                    a2 = a2g[j]
                    for k in range(HH // 16):
                        p = rows_v[i, pl.ds(k * 16, 16)]
                        m = jnp.maximum(
                            p + a0 * w[0][k] + a1 * w[1][k] + a2 * w[2][k],
                            0.0)
                        rows_v[i, pl.ds(k * 16, 16)] = m
                return carry2

            lax.fori_loop(0, CE // 16, grp, 0)

            for g in range(G):
                pltpu.sync_copy(rows_v.at[pl.ds(g * 64, 64)],
                                agg_sh.at[idxd_g[g]], add=True)
            return carry

        lax.fori_loop(0, NCHUNK, chunk, 0)
        plsc.subcore_barrier()
        pltpu.sync_copy(agg_sh.at[pl.ds(n0, 624)],
                        agg_ref.at[pl.ds(coff + n0, 624)])

        @pl.when(s == 0)
        def _():
            pltpu.sync_copy(agg_sh.at[pl.ds(N - 16, 16)],
                            agg_ref.at[pl.ds(coff + N - 16, 16)])

    return body(pflat, src1, dst1, attr1, wme1)


def _dot(a, b):
    return jnp.dot(a, b, preferred_element_type=jnp.float32)


BR = 1000
_NB = N // BR


def _row_spec(cols):
    return pl.BlockSpec((BR, cols), lambda i: (i, 0))


def _agg_spec():
    return pl.BlockSpec((2, BR, HH), lambda i: (0, i, 0))


def _full_spec(shape):
    nd = len(shape)
    return pl.BlockSpec(shape, lambda i, _nd=nd: (0,) * nd)


def _encoder_call(x, We1, s1, t1, We2, s2, t2, Wmh0, bm0):
    def body(x_ref, we1_ref, s1_ref, t1_ref, we2_ref, s2_ref, t2_ref,
             wm_ref, bm_ref, h_ref, p_ref):
        h1 = jnp.maximum(_dot(x_ref[...], we1_ref[...]) * s1_ref[...]
                         + t1_ref[...], 0.0)
        h2 = jnp.maximum(_dot(h1, we2_ref[...]) * s2_ref[...]
                         + t2_ref[...], 0.0)
        h_ref[...] = h2
        p = _dot(h2, wm_ref[...]) + bm_ref[...]
        p_ref[0] = p[:, :HH]
        p_ref[1] = p[:, HH:]

    return pl.pallas_call(
        body,
        grid=(_NB,),
        in_specs=[_row_spec(H)] + [_full_spec(a.shape) for a in
                                   (We1, s1, t1, We2, s2, t2, Wmh0, bm0)],
        out_specs=(_row_spec(H), _agg_spec()),
        out_shape=(jax.ShapeDtypeStruct((N, H), jnp.float32),
                   jax.ShapeDtypeStruct((2, N, HH), jnp.float32)),
    )(x, We1, s1, t1, We2, s2, t2, Wmh0, bm0)


def _update_call(h, agg, Wuh, Wua, bu, Wmh_next, bm_next):
    def body(h_ref, agg_ref, wuh_ref, wua_ref, bu_ref, wm_ref, bm_ref,
             hn_ref, p_ref):
        h_in = h_ref[...]
        hn = (h_in + _dot(h_in, wuh_ref[...])
              + _dot(agg_ref[0], wua_ref[0])
              + _dot(agg_ref[1], wua_ref[1]) + bu_ref[...])
        hn_ref[...] = hn
        p = _dot(hn, wm_ref[...]) + bm_ref[...]
        p_ref[0] = p[:, :HH]
        p_ref[1] = p[:, HH:]

    return pl.pallas_call(
        body,
        grid=(_NB,),
        in_specs=[_row_spec(H), _agg_spec()] + [
            _full_spec(a.shape) for a in (Wuh, Wua, bu, Wmh_next, bm_next)],
        out_specs=(_row_spec(H), _agg_spec()),
        out_shape=(jax.ShapeDtypeStruct((N, H), jnp.float32),
                   jax.ShapeDtypeStruct((2, N, HH), jnp.float32)),
    )(h, agg, Wuh, Wua, bu, Wmh_next, bm_next)


def _final_call(h, agg, Wuh, Wua, bu, Wd1, bd1, Wd2, bd2):
    def body(h_ref, agg_ref, wuh_ref, wua_ref, bu_ref, wd1_ref, bd1_ref,
             wd2_ref, bd2_ref, out_ref):
        h_in = h_ref[...]
        hn = (h_in + _dot(h_in, wuh_ref[...])
              + _dot(agg_ref[0], wua_ref[0])
              + _dot(agg_ref[1], wua_ref[1]) + bu_ref[...])
        d1 = jnp.maximum(_dot(hn, wd1_ref[...]) + bd1_ref[...], 0.0)
        out_ref[...] = _dot(d1, wd2_ref[...]) + bd2_ref[...]

    return pl.pallas_call(
        body,
        grid=(_NB,),
        in_specs=[_row_spec(H), _agg_spec()] + [
            _full_spec(a.shape) for a in (Wuh, Wua, bu, Wd1, bd1, Wd2, bd2)],
        out_specs=_row_spec(3),
        out_shape=jax.ShapeDtypeStruct((N, 3), jnp.float32),
    )(h, agg, Wuh, Wua, bu, Wd1, bd1, Wd2, bd2)


def kernel(x, edge_index, edge_attr, We1, be1, g1, b1, m1, v1, We2, be2,
           g2, b2, m2, v2, Wm, bm, Wu, bu, Wd1, bd1, Wd2, bd2):
    L = Wm.shape[0]
    s1 = g1 / jnp.sqrt(v1 + 1e-5)
    t1 = (be1 - m1) * s1 + b1
    s2 = g2 / jnp.sqrt(v2 + 1e-5)
    t2 = (be2 - m2) * s2 + b2

    src1 = edge_index[0]
    dst1 = edge_index[1]
    attr1 = (edge_attr.T.reshape(3 * E)
             .astype(jnp.bfloat16).astype(jnp.float32))

    Wmh = Wm[:, :H, :]
    Wme = Wm[:, H:, :]
    wme = (jnp.concatenate([Wme[:, :, :HH], Wme[:, :, HH:]], axis=1)
           .reshape(L, 6 * HH).astype(jnp.bfloat16).astype(jnp.float32))
    Wuh = Wu[:, :H, :]
    Wua = Wu[:, H:, :].reshape(L, 2, HH, H)

    h, p = _encoder_call(x, We1, s1, t1, We2, s2, t2, Wmh[0], bm[0])
    for l in range(L):
        aggflat = _edge_kernel(p.reshape(2 * N, HH), src1, dst1, attr1,
                               wme[l])
        agg = aggflat.reshape(2, N, HH)
        if l + 1 < L:
            h, p = _update_call(h, agg, Wuh[l], Wua[l], bu[l],
                                Wmh[l + 1], bm[l + 1])
        else:
            pred = _final_call(h, agg, Wuh[l], Wua[l], bu[l],
                               Wd1, bd1, Wd2, bd2)
    return pred


# speedup vs baseline: 4.1940x; 1.6131x over previous
import functools

import jax
import jax.numpy as jnp
from jax import lax
from jax.experimental import pallas as pl
from jax.experimental.pallas import tpu as pltpu
from jax.experimental.pallas import tpu_sc as plsc

N = 10000
E = 640000
H = 128
HH = 64
NSUB = 16
NCORE = 2
EPS = E // NSUB
G = 5
CE = G * 64
NCHUNK = EPS // CE
NPS = N // NSUB


def _edge_kernel(pflat, srcoff, dst1, attrc, wme1):
    mesh = plsc.VectorSubcoreMesh(core_axis_name="c", subcore_axis_name="s")

    @functools.partial(
        pl.kernel,
        out_type=jax.ShapeDtypeStruct((NCORE * N, HH), jnp.float32),
        mesh=mesh,
        compiler_params=pltpu.CompilerParams(use_tc_tiling_on_sc=False),
        scratch_types=(
            [pltpu.VMEM((64,), jnp.int32) for _ in range(G)]
            + [
                pltpu.VMEM((CE,), jnp.int32),
                pltpu.VMEM((3 * CE,), jnp.float32),
                pltpu.VMEM((CE, HH), jnp.float32),
                pltpu.VMEM((3 * HH,), jnp.float32),
                pltpu.VMEM_SHARED((N, HH), jnp.float32),
                pltpu.SemaphoreType.DMA,
                pltpu.SemaphoreType.DMA,
                pltpu.SemaphoreType.DMA,
            ]
        ),
    )
    def body(p_ref, src_ref, dst_ref, attr_ref, wme_ref, agg_ref, *scr):
        idxd_g = scr[:G]
        idxs_v, attr_v, rows_v, wm_v, agg_sh, semi, semg, sems = scr[G:]
        c = lax.axis_index("c")
        s = lax.axis_index("s")

        pltpu.sync_copy(wme_ref.at[pl.ds(c * 3 * HH, 3 * HH)], wm_v)

        zero = jnp.zeros((16,), jnp.float32)

        def zrow(i, carry):
            for k in range(HH // 16):
                rows_v[i, pl.ds(k * 16, 16)] = zero
            return carry

        lax.fori_loop(0, CE, zrow, 0)
        n0 = s * 624
        pltpu.sync_copy(rows_v, agg_sh.at[pl.ds(n0, CE)])
        pltpu.sync_copy(rows_v.at[pl.ds(0, 624 - CE)],
                        agg_sh.at[pl.ds(n0 + CE, 624 - CE)])

        @pl.when(s == 0)
        def _():
            pltpu.sync_copy(rows_v.at[pl.ds(0, 16)],
                            agg_sh.at[pl.ds(N - 16, 16)])

        plsc.subcore_barrier()

        w = [[wm_v[pl.ds(j * HH + k * 16, 16)] for k in range(HH // 16)]
             for j in range(3)]
        coff = c * N

        def chunk(t, carry):
            e0 = s * EPS + t * CE

            @pl.when(t > 0)
            def _():
                pltpu.make_async_copy(
                    p_ref.at[pl.ds(0, CE)], rows_v, sems).wait()

            di = [
                pltpu.async_copy(src_ref.at[pl.ds(c * E + e0, CE)],
                                 idxs_v, semi),
                pltpu.async_copy(attr_ref.at[pl.ds(3 * e0, 3 * CE)],
                                 attr_v, semi),
            ] + [
                pltpu.async_copy(dst_ref.at[pl.ds(e0 + g * 64, 64)],
                                 idxd_g[g], semi)
                for g in range(G)
            ]
            for d in di:
                d.wait()

            dg = [
                pltpu.async_copy(p_ref.at[idxs_v.at[pl.ds(g * 64, 64)]],
                                 rows_v.at[pl.ds(g * 64, 64)], semg)
                for g in range(G)
            ]
            for d in dg:
                d.wait()

            def grp(q, carry2):
                i0 = q * 16
                a0g = attr_v[pl.ds(i0, 16)]
                a1g = attr_v[pl.ds(CE + i0, 16)]
                a2g = attr_v[pl.ds(2 * CE + i0, 16)]
                for j in range(16):
                    i = i0 + j
                    a0 = a0g[j]
                    a1 = a1g[j]
                    a2 = a2g[j]
                    for k in range(HH // 16):
                        p = rows_v[i, pl.ds(k * 16, 16)]
                        m = jnp.maximum(
                            p + a0 * w[0][k] + a1 * w[1][k] + a2 * w[2][k],
                            0.0)
                        rows_v[i, pl.ds(k * 16, 16)] = m
                return carry2

            lax.fori_loop(0, CE // 16, grp, 0)

            for g in range(G):
                pltpu.async_copy(rows_v.at[pl.ds(g * 64, 64)],
                                 agg_sh.at[idxd_g[g]], sems, add=True)
            return carry

        lax.fori_loop(0, NCHUNK, chunk, 0)
        pltpu.make_async_copy(p_ref.at[pl.ds(0, CE)], rows_v, sems).wait()
        plsc.subcore_barrier()
        pltpu.sync_copy(agg_sh.at[pl.ds(n0, 624)],
                        agg_ref.at[pl.ds(coff + n0, 624)])

        @pl.when(s == 0)
        def _():
            pltpu.sync_copy(agg_sh.at[pl.ds(N - 16, 16)],
                            agg_ref.at[pl.ds(coff + N - 16, 16)])

    return body(pflat, srcoff, dst1, attrc, wme1)


def _dot(a, b):
    return jnp.dot(a, b, preferred_element_type=jnp.float32)


BR = 1000
_NB = N // BR


def _row_spec(cols):
    return pl.BlockSpec((BR, cols), lambda i: (i, 0))


def _agg_spec():
    return pl.BlockSpec((2, BR, HH), lambda i: (0, i, 0))


def _full_spec(shape):
    nd = len(shape)
    return pl.BlockSpec(shape, lambda i, _nd=nd: (0,) * nd)


def _encoder_call(x, We1, s1, t1, We2, s2, t2, Wmh0, bm0):
    def body(x_ref, we1_ref, s1_ref, t1_ref, we2_ref, s2_ref, t2_ref,
             wm_ref, bm_ref, h_ref, p_ref):
        h1 = jnp.maximum(_dot(x_ref[...], we1_ref[...]) * s1_ref[...]
                         + t1_ref[...], 0.0)
        h2 = jnp.maximum(_dot(h1, we2_ref[...]) * s2_ref[...]
                         + t2_ref[...], 0.0)
        h_ref[...] = h2
        p = _dot(h2, wm_ref[...]) + bm_ref[...]
        p_ref[0] = p[:, :HH]
        p_ref[1] = p[:, HH:]

    return pl.pallas_call(
        body,
        grid=(_NB,),
        in_specs=[_row_spec(H)] + [_full_spec(a.shape) for a in
                                   (We1, s1, t1, We2, s2, t2, Wmh0, bm0)],
        out_specs=(_row_spec(H), _agg_spec()),
        out_shape=(jax.ShapeDtypeStruct((N, H), jnp.float32),
                   jax.ShapeDtypeStruct((2, N, HH), jnp.float32)),
    )(x, We1, s1, t1, We2, s2, t2, Wmh0, bm0)


def _update_call(h, agg, Wuh, Wua, bu, Wmh_next, bm_next):
    def body(h_ref, agg_ref, wuh_ref, wua_ref, bu_ref, wm_ref, bm_ref,
             hn_ref, p_ref):
        h_in = h_ref[...]
        hn = (h_in + _dot(h_in, wuh_ref[...])
              + _dot(agg_ref[0], wua_ref[0])
              + _dot(agg_ref[1], wua_ref[1]) + bu_ref[...])
        hn_ref[...] = hn
        p = _dot(hn, wm_ref[...]) + bm_ref[...]
        p_ref[0] = p[:, :HH]
        p_ref[1] = p[:, HH:]

    return pl.pallas_call(
        body,
        grid=(_NB,),
        in_specs=[_row_spec(H), _agg_spec()] + [
            _full_spec(a.shape) for a in (Wuh, Wua, bu, Wmh_next, bm_next)],
        out_specs=(_row_spec(H), _agg_spec()),
        out_shape=(jax.ShapeDtypeStruct((N, H), jnp.float32),
                   jax.ShapeDtypeStruct((2, N, HH), jnp.float32)),
    )(h, agg, Wuh, Wua, bu, Wmh_next, bm_next)


def _final_call(h, agg, Wuh, Wua, bu, Wd1, bd1, Wd2, bd2):
    def body(h_ref, agg_ref, wuh_ref, wua_ref, bu_ref, wd1_ref, bd1_ref,
             wd2_ref, bd2_ref, out_ref):
        h_in = h_ref[...]
        hn = (h_in + _dot(h_in, wuh_ref[...])
              + _dot(agg_ref[0], wua_ref[0])
              + _dot(agg_ref[1], wua_ref[1]) + bu_ref[...])
        d1 = jnp.maximum(_dot(hn, wd1_ref[...]) + bd1_ref[...], 0.0)
        out_ref[...] = _dot(d1, wd2_ref[...]) + bd2_ref[...]

    return pl.pallas_call(
        body,
        grid=(_NB,),
        in_specs=[_row_spec(H), _agg_spec()] + [
            _full_spec(a.shape) for a in (Wuh, Wua, bu, Wd1, bd1, Wd2, bd2)],
        out_specs=_row_spec(3),
        out_shape=jax.ShapeDtypeStruct((N, 3), jnp.float32),
    )(h, agg, Wuh, Wua, bu, Wd1, bd1, Wd2, bd2)


def kernel(x, edge_index, edge_attr, We1, be1, g1, b1, m1, v1, We2, be2,
           g2, b2, m2, v2, Wm, bm, Wu, bu, Wd1, bd1, Wd2, bd2):
    L = Wm.shape[0]
    s1 = g1 / jnp.sqrt(v1 + 1e-5)
    t1 = (be1 - m1) * s1 + b1
    s2 = g2 / jnp.sqrt(v2 + 1e-5)
    t2 = (be2 - m2) * s2 + b2

    src = edge_index[0]
    srcoff = jnp.concatenate([src, src + N])
    dst1 = edge_index[1]
    attrc = (edge_attr.T.reshape(3, E // CE, CE).transpose(1, 0, 2)
             .reshape(3 * E).astype(jnp.bfloat16).astype(jnp.float32))

    Wmh = Wm[:, :H, :]
    Wme = Wm[:, H:, :]
    wme = (jnp.concatenate([Wme[:, :, :HH], Wme[:, :, HH:]], axis=1)
           .reshape(L, 6 * HH).astype(jnp.bfloat16).astype(jnp.float32))
    Wuh = Wu[:, :H, :]
    Wua = Wu[:, H:, :].reshape(L, 2, HH, H)

    h, p = _encoder_call(x, We1, s1, t1, We2, s2, t2, Wmh[0], bm[0])
    for l in range(L):
        aggflat = _edge_kernel(p.reshape(2 * N, HH), srcoff, dst1, attrc,
                               wme[l])
        agg = aggflat.reshape(2, N, HH)
        if l + 1 < L:
            h, p = _update_call(h, agg, Wuh[l], Wua[l], bu[l],
                                Wmh[l + 1], bm[l + 1])
        else:
            pred = _final_call(h, agg, Wuh[l], Wua[l], bu[l],
                               Wd1, bd1, Wd2, bd2)
    return pred


# speedup vs baseline: 4.8576x; 1.1582x over previous
import functools

import jax
import jax.numpy as jnp
from jax import lax
from jax.experimental import pallas as pl
from jax.experimental.pallas import tpu as pltpu
from jax.experimental.pallas import tpu_sc as plsc

N = 10000
E = 640000
H = 128
HH = 64
NSUB = 16
NCORE = 2
EPS = E // NSUB
G = 5
CE = G * 64
NCHUNK = EPS // CE
NPS = N // NSUB


def _edge_kernel(pflat, srcoff, dst1, attrc, wme1):
    mesh = plsc.VectorSubcoreMesh(core_axis_name="c", subcore_axis_name="s")

    @functools.partial(
        pl.kernel,
        out_type=jax.ShapeDtypeStruct((NCORE * N, HH), jnp.float32),
        mesh=mesh,
        compiler_params=pltpu.CompilerParams(use_tc_tiling_on_sc=False),
        scratch_types=(
            [pltpu.VMEM((64,), jnp.int32) for _ in range(2 * G)]
            + [pltpu.VMEM((CE,), jnp.int32) for _ in range(2)]
            + [pltpu.VMEM((3 * CE,), jnp.float32) for _ in range(2)]
            + [pltpu.VMEM((CE, HH), jnp.float32) for _ in range(2)]
            + [
                pltpu.VMEM((3 * HH,), jnp.float32),
                pltpu.VMEM_SHARED((N, HH), jnp.float32),
            ]
            + [pltpu.SemaphoreType.DMA for _ in range(8)]
        ),
    )
    def body(p_ref, src_ref, dst_ref, attr_ref, wme_ref, agg_ref, *scr):
        idxd = [scr[:G], scr[G:2 * G]]
        idxs = scr[2 * G:2 * G + 2]
        attr = scr[2 * G + 2:2 * G + 4]
        rows = scr[2 * G + 4:2 * G + 6]
        wm_v, agg_sh = scr[2 * G + 6:2 * G + 8]
        semi = scr[2 * G + 8:2 * G + 10]
        semid = scr[2 * G + 10:2 * G + 12]
        semg = scr[2 * G + 12:2 * G + 14]
        sems = scr[2 * G + 14:2 * G + 16]
        c = lax.axis_index("c")
        s = lax.axis_index("s")

        pltpu.sync_copy(wme_ref.at[pl.ds(c * 3 * HH, 3 * HH)], wm_v)

        zero = jnp.zeros((16,), jnp.float32)

        def zrow(i, carry):
            for k in range(HH // 16):
                rows[0][i, pl.ds(k * 16, 16)] = zero
            return carry

        lax.fori_loop(0, CE, zrow, 0)
        n0 = s * 624
        pltpu.sync_copy(rows[0], agg_sh.at[pl.ds(n0, CE)])
        pltpu.sync_copy(rows[0].at[pl.ds(0, 624 - CE)],
                        agg_sh.at[pl.ds(n0 + CE, 624 - CE)])

        @pl.when(s == 0)
        def _():
            pltpu.sync_copy(rows[0].at[pl.ds(0, 16)],
                            agg_sh.at[pl.ds(N - 16, 16)])

        plsc.subcore_barrier()

        w = [[wm_v[pl.ds(j * HH + k * 16, 16)] for k in range(HH // 16)]
             for j in range(3)]
        coff = c * N

        def fire_isa(b, cc):
            e0 = s * EPS + cc * CE
            pltpu.async_copy(src_ref.at[pl.ds(c * E + e0, CE)],
                             idxs[b], semi[b])
            pltpu.async_copy(attr_ref.at[pl.ds(3 * e0, 3 * CE)],
                             attr[b], semi[b])

        def wait_isa(b):
            pltpu.make_async_copy(src_ref.at[pl.ds(0, CE)],
                                  idxs[b], semi[b]).wait()
            pltpu.make_async_copy(attr_ref.at[pl.ds(0, 3 * CE)],
                                  attr[b], semi[b]).wait()

        def fire_id(b, cc):
            e0 = s * EPS + cc * CE
            for g in range(G):
                pltpu.async_copy(dst_ref.at[pl.ds(e0 + g * 64, 64)],
                                 idxd[b][g], semid[b])

        def wait_id(b):
            for g in range(G):
                pltpu.make_async_copy(dst_ref.at[pl.ds(0, 64)],
                                      idxd[b][g], semid[b]).wait()

        def fire_g(b):
            for g in range(G):
                pltpu.async_copy(p_ref.at[idxs[b].at[pl.ds(g * 64, 64)]],
                                 rows[b].at[pl.ds(g * 64, 64)], semg[b])

        def wait_g(b):
            pltpu.make_async_copy(p_ref.at[pl.ds(0, CE)],
                                  rows[b], semg[b]).wait()

        def fire_s(b):
            for g in range(G):
                pltpu.async_copy(rows[b].at[pl.ds(g * 64, 64)],
                                 agg_sh.at[idxd[b][g]], sems[b], add=True)

        def drain_s(b):
            pltpu.make_async_copy(p_ref.at[pl.ds(0, CE)],
                                  rows[b], sems[b]).wait()

        def compute(b):
            def grp(q, carry2):
                i0 = q * 16
                a0g = attr[b][pl.ds(i0, 16)]
                a1g = attr[b][pl.ds(CE + i0, 16)]
                a2g = attr[b][pl.ds(2 * CE + i0, 16)]
                for j in range(16):
                    i = i0 + j
                    a0 = a0g[j]
                    a1 = a1g[j]
                    a2 = a2g[j]
                    for k in range(HH // 16):
                        p = rows[b][i, pl.ds(k * 16, 16)]
                        m = jnp.maximum(
                            p + a0 * w[0][k] + a1 * w[1][k] + a2 * w[2][k],
                            0.0)
                        rows[b][i, pl.ds(k * 16, 16)] = m
                return carry2

            lax.fori_loop(0, CE // 16, grp, 0)

        def position(b, cc, has_next):
            wait_g(b)
            compute(b)
            wait_id(b)
            fire_s(b)
            if has_next:
                o = b ^ 1
                wait_isa(o)

                @pl.when(cc > 0)
                def _():
                    drain_s(o)

                fire_g(o)
                fire_id(o, cc + 1)

                @pl.when(cc + 2 < NCHUNK)
                def _():
                    fire_isa(b, cc + 2)

        fire_isa(0, 0)
        fire_id(0, 0)
        wait_isa(0)
        fire_g(0)
        fire_isa(1, 1)

        def pipe(t, carry):
            position(0, 2 * t, True)
            position(1, 2 * t + 1, True)
            return carry

        lax.fori_loop(0, (NCHUNK - 1) // 2, pipe, 0)
        position(0, NCHUNK - 1, False)
        drain_s(1)
        drain_s(0)
        plsc.subcore_barrier()
        pltpu.sync_copy(agg_sh.at[pl.ds(n0, 624)],
                        agg_ref.at[pl.ds(coff + n0, 624)])

        @pl.when(s == 0)
        def _():
            pltpu.sync_copy(agg_sh.at[pl.ds(N - 16, 16)],
                            agg_ref.at[pl.ds(coff + N - 16, 16)])

    return body(pflat, srcoff, dst1, attrc, wme1)


def _dot(a, b):
    return jnp.dot(a, b, preferred_element_type=jnp.float32)


BR = 1000
_NB = N // BR


def _row_spec(cols):
    return pl.BlockSpec((BR, cols), lambda i: (i, 0))


def _agg_spec():
    return pl.BlockSpec((2, BR, HH), lambda i: (0, i, 0))


def _full_spec(shape):
    nd = len(shape)
    return pl.BlockSpec(shape, lambda i, _nd=nd: (0,) * nd)


def _encoder_call(x, We1, s1, t1, We2, s2, t2, Wmh0, bm0):
    def body(x_ref, we1_ref, s1_ref, t1_ref, we2_ref, s2_ref, t2_ref,
             wm_ref, bm_ref, h_ref, p_ref):
        h1 = jnp.maximum(_dot(x_ref[...], we1_ref[...]) * s1_ref[...]
                         + t1_ref[...], 0.0)
        h2 = jnp.maximum(_dot(h1, we2_ref[...]) * s2_ref[...]
                         + t2_ref[...], 0.0)
        h_ref[...] = h2
        p = _dot(h2, wm_ref[...]) + bm_ref[...]
        p_ref[0] = p[:, :HH]
        p_ref[1] = p[:, HH:]

    return pl.pallas_call(
        body,
        grid=(_NB,),
        in_specs=[_row_spec(H)] + [_full_spec(a.shape) for a in
                                   (We1, s1, t1, We2, s2, t2, Wmh0, bm0)],
        out_specs=(_row_spec(H), _agg_spec()),
        out_shape=(jax.ShapeDtypeStruct((N, H), jnp.float32),
                   jax.ShapeDtypeStruct((2, N, HH), jnp.float32)),
    )(x, We1, s1, t1, We2, s2, t2, Wmh0, bm0)


def _update_call(h, agg, Wuh, Wua, bu, Wmh_next, bm_next):
    def body(h_ref, agg_ref, wuh_ref, wua_ref, bu_ref, wm_ref, bm_ref,
             hn_ref, p_ref):
        h_in = h_ref[...]
        hn = (h_in + _dot(h_in, wuh_ref[...])
              + _dot(agg_ref[0], wua_ref[0])
              + _dot(agg_ref[1], wua_ref[1]) + bu_ref[...])
        hn_ref[...] = hn
        p = _dot(hn, wm_ref[...]) + bm_ref[...]
        p_ref[0] = p[:, :HH]
        p_ref[1] = p[:, HH:]

    return pl.pallas_call(
        body,
        grid=(_NB,),
        in_specs=[_row_spec(H), _agg_spec()] + [
            _full_spec(a.shape) for a in (Wuh, Wua, bu, Wmh_next, bm_next)],
        out_specs=(_row_spec(H), _agg_spec()),
        out_shape=(jax.ShapeDtypeStruct((N, H), jnp.float32),
                   jax.ShapeDtypeStruct((2, N, HH), jnp.float32)),
    )(h, agg, Wuh, Wua, bu, Wmh_next, bm_next)


def _final_call(h, agg, Wuh, Wua, bu, Wd1, bd1, Wd2, bd2):
    def body(h_ref, agg_ref, wuh_ref, wua_ref, bu_ref, wd1_ref, bd1_ref,
             wd2_ref, bd2_ref, out_ref):
        h_in = h_ref[...]
        hn = (h_in + _dot(h_in, wuh_ref[...])
              + _dot(agg_ref[0], wua_ref[0])
              + _dot(agg_ref[1], wua_ref[1]) + bu_ref[...])
        d1 = jnp.maximum(_dot(hn, wd1_ref[...]) + bd1_ref[...], 0.0)
        out_ref[...] = _dot(d1, wd2_ref[...]) + bd2_ref[...]

    return pl.pallas_call(
        body,
        grid=(_NB,),
        in_specs=[_row_spec(H), _agg_spec()] + [
            _full_spec(a.shape) for a in (Wuh, Wua, bu, Wd1, bd1, Wd2, bd2)],
        out_specs=_row_spec(3),
        out_shape=jax.ShapeDtypeStruct((N, 3), jnp.float32),
    )(h, agg, Wuh, Wua, bu, Wd1, bd1, Wd2, bd2)


def kernel(x, edge_index, edge_attr, We1, be1, g1, b1, m1, v1, We2, be2,
           g2, b2, m2, v2, Wm, bm, Wu, bu, Wd1, bd1, Wd2, bd2):
    L = Wm.shape[0]
    s1 = g1 / jnp.sqrt(v1 + 1e-5)
    t1 = (be1 - m1) * s1 + b1
    s2 = g2 / jnp.sqrt(v2 + 1e-5)
    t2 = (be2 - m2) * s2 + b2

    src = edge_index[0]
    srcoff = jnp.concatenate([src, src + N])
    dst1 = edge_index[1]
    attrc = (edge_attr.T.reshape(3, E // CE, CE).transpose(1, 0, 2)
             .reshape(3 * E).astype(jnp.bfloat16).astype(jnp.float32))

    Wmh = Wm[:, :H, :]
    Wme = Wm[:, H:, :]
    wme = (jnp.concatenate([Wme[:, :, :HH], Wme[:, :, HH:]], axis=1)
           .reshape(L, 6 * HH).astype(jnp.bfloat16).astype(jnp.float32))
    Wuh = Wu[:, :H, :]
    Wua = Wu[:, H:, :].reshape(L, 2, HH, H)

    h, p = _encoder_call(x, We1, s1, t1, We2, s2, t2, Wmh[0], bm[0])
    for l in range(L):
        aggflat = _edge_kernel(p.reshape(2 * N, HH), srcoff, dst1, attrc,
                               wme[l])
        agg = aggflat.reshape(2, N, HH)
        if l + 1 < L:
            h, p = _update_call(h, agg, Wuh[l], Wua[l], bu[l],
                                Wmh[l + 1], bm[l + 1])
        else:
            pred = _final_call(h, agg, Wuh[l], Wua[l], bu[l],
                               Wd1, bd1, Wd2, bd2)
    return pred


# speedup vs baseline: 7.2388x; 1.4902x over previous
import functools

import jax
import jax.numpy as jnp
from jax import lax
from jax.experimental import pallas as pl
from jax.experimental.pallas import tpu as pltpu
from jax.experimental.pallas import tpu_sc as plsc

N = 10000
E = 640000
H = 128
HH = 64
NSUB = 16
NCORE = 2
EPS = E // NSUB
G = 5
CE = G * 64
NCHUNK = EPS // CE
NPS = N // NSUB


def _edge_kernel(pflat, srcoff, dst1, attrc, wme1):
    mesh = plsc.VectorSubcoreMesh(core_axis_name="c", subcore_axis_name="s")

    @functools.partial(
        pl.kernel,
        out_type=jax.ShapeDtypeStruct((NCORE * N, HH), jnp.float32),
        mesh=mesh,
        compiler_params=pltpu.CompilerParams(use_tc_tiling_on_sc=False),
        scratch_types=(
            [pltpu.VMEM((64,), jnp.int32) for _ in range(3 * G)]
            + [pltpu.VMEM((CE,), jnp.int32) for _ in range(3)]
            + [pltpu.VMEM((3 * CE,), jnp.float32) for _ in range(3)]
            + [pltpu.VMEM((CE, HH), jnp.float32) for _ in range(3)]
            + [
                pltpu.VMEM((3 * HH,), jnp.float32),
                pltpu.VMEM_SHARED((N, HH), jnp.float32),
            ]
            + [pltpu.SemaphoreType.DMA for _ in range(12)]
        ),
    )
    def body(p_ref, src_ref, dst_ref, attr_ref, wme_ref, agg_ref, *scr):
        idxd = [scr[:G], scr[G:2 * G], scr[2 * G:3 * G]]
        o = 3 * G
        idxs = scr[o:o + 3]
        attr = scr[o + 3:o + 6]
        rows = scr[o + 6:o + 9]
        wm_v, agg_sh = scr[o + 9:o + 11]
        semi = scr[o + 11:o + 14]
        semid = scr[o + 14:o + 17]
        semg = scr[o + 17:o + 20]
        sems = scr[o + 20:o + 23]
        c = lax.axis_index("c")
        s = lax.axis_index("s")

        pltpu.sync_copy(wme_ref.at[pl.ds(c * 3 * HH, 3 * HH)], wm_v)

        zero = jnp.zeros((16,), jnp.float32)

        def zrow(i, carry):
            for k in range(HH // 16):
                rows[0][i, pl.ds(k * 16, 16)] = zero
            return carry

        lax.fori_loop(0, CE, zrow, 0)
        n0 = s * 624
        pltpu.sync_copy(rows[0], agg_sh.at[pl.ds(n0, CE)])
        pltpu.sync_copy(rows[0].at[pl.ds(0, 624 - CE)],
                        agg_sh.at[pl.ds(n0 + CE, 624 - CE)])

        @pl.when(s == 0)
        def _():
            pltpu.sync_copy(rows[0].at[pl.ds(0, 16)],
                            agg_sh.at[pl.ds(N - 16, 16)])

        plsc.subcore_barrier()

        w = [[wm_v[pl.ds(j * HH + k * 16, 16)] for k in range(HH // 16)]
             for j in range(3)]
        coff = c * N

        def fire_isa(b, cc):
            e0 = s * EPS + cc * CE
            pltpu.async_copy(src_ref.at[pl.ds(c * E + e0, CE)],
                             idxs[b], semi[b])
            pltpu.async_copy(attr_ref.at[pl.ds(3 * e0, 3 * CE)],
                             attr[b], semi[b])

        def wait_isa(b):
            pltpu.make_async_copy(src_ref.at[pl.ds(0, CE)],
                                  idxs[b], semi[b]).wait()
            pltpu.make_async_copy(attr_ref.at[pl.ds(0, 3 * CE)],
                                  attr[b], semi[b]).wait()

        def fire_id(b, cc):
            e0 = s * EPS + cc * CE
            for g in range(G):
                pltpu.async_copy(dst_ref.at[pl.ds(e0 + g * 64, 64)],
                                 idxd[b][g], semid[b])

        def wait_id(b):
            for g in range(G):
                pltpu.make_async_copy(dst_ref.at[pl.ds(0, 64)],
                                      idxd[b][g], semid[b]).wait()

        def fire_g(b):
            for g in range(G):
                pltpu.async_copy(p_ref.at[idxs[b].at[pl.ds(g * 64, 64)]],
                                 rows[b].at[pl.ds(g * 64, 64)], semg[b])

        def wait_g(b):
            pltpu.make_async_copy(p_ref.at[pl.ds(0, CE)],
                                  rows[b], semg[b]).wait()

        def fire_s(b):
            for g in range(G):
                pltpu.async_copy(rows[b].at[pl.ds(g * 64, 64)],
                                 agg_sh.at[idxd[b][g]], sems[b], add=True)

        def drain_s(b):
            pltpu.make_async_copy(p_ref.at[pl.ds(0, CE)],
                                  rows[b], sems[b]).wait()

        def compute(b):
            def grp(q, carry2):
                i0 = q * 16
                a0g = attr[b][pl.ds(i0, 16)]
                a1g = attr[b][pl.ds(CE + i0, 16)]
                a2g = attr[b][pl.ds(2 * CE + i0, 16)]
                for j in range(16):
                    i = i0 + j
                    a0 = a0g[j]
                    a1 = a1g[j]
                    a2 = a2g[j]
                    for k in range(HH // 16):
                        p = rows[b][i, pl.ds(k * 16, 16)]
                        m = jnp.maximum(
                            (p + a0 * w[0][k])
                            + (a1 * w[1][k] + a2 * w[2][k]), 0.0)
                        rows[b][i, pl.ds(k * 16, 16)] = m
                return carry2

            lax.fori_loop(0, CE // 16, grp, 0)

        def position(b, cc, fire_next, fire_next2, guard0=False):
            b1 = (b + 1) % 3
            b2 = (b + 2) % 3
            wait_g(b)
            if fire_next:
                wait_isa(b1)
                fire_g(b1)
            wait_id(b)
            compute(b)
            fire_s(b)
            if guard0:
                @pl.when(cc > 0)
                def _():
                    drain_s(b2)
            else:
                drain_s(b2)
            if fire_next2:
                fire_isa(b2, cc + 2)
                fire_id(b2, cc + 2)

        fire_isa(0, 0)
        fire_id(0, 0)
        wait_isa(0)
        fire_g(0)
        fire_isa(1, 1)
        fire_id(1, 1)

        def pipe(t, carry):
            cc = 3 * t
            position(0, cc, True, True, guard0=True)
            position(1, cc + 1, True, True)
            position(2, cc + 2, True, True)
            return carry

        lax.fori_loop(0, (NCHUNK - 2) // 3, pipe, 0)
        position(0, NCHUNK - 2, True, False)
        position(1, NCHUNK - 1, False, False)
        drain_s(1)
        plsc.subcore_barrier()
        pltpu.sync_copy(agg_sh.at[pl.ds(n0, 624)],
                        agg_ref.at[pl.ds(coff + n0, 624)])

        @pl.when(s == 0)
        def _():
            pltpu.sync_copy(agg_sh.at[pl.ds(N - 16, 16)],
                            agg_ref.at[pl.ds(coff + N - 16, 16)])

    return body(pflat, srcoff, dst1, attrc, wme1)


def _dot(a, b):
    return jnp.dot(a, b, preferred_element_type=jnp.float32)


BR = 1000
_NB = N // BR


def _row_spec(cols):
    return pl.BlockSpec((BR, cols), lambda i: (i, 0))


def _agg_spec():
    return pl.BlockSpec((2, BR, HH), lambda i: (0, i, 0))


def _full_spec(shape):
    nd = len(shape)
    return pl.BlockSpec(shape, lambda i, _nd=nd: (0,) * nd)


def _encoder_call(x, We1, s1, t1, We2, s2, t2, Wmh0, bm0):
    def body(x_ref, we1_ref, s1_ref, t1_ref, we2_ref, s2_ref, t2_ref,
             wm_ref, bm_ref, h_ref, p_ref):
        h1 = jnp.maximum(_dot(x_ref[...], we1_ref[...]) * s1_ref[...]
                         + t1_ref[...], 0.0)
        h2 = jnp.maximum(_dot(h1, we2_ref[...]) * s2_ref[...]
                         + t2_ref[...], 0.0)
        h_ref[...] = h2
        p = _dot(h2, wm_ref[...]) + bm_ref[...]
        p_ref[0] = p[:, :HH]
        p_ref[1] = p[:, HH:]

    return pl.pallas_call(
        body,
        grid=(_NB,),
        in_specs=[_row_spec(H)] + [_full_spec(a.shape) for a in
                                   (We1, s1, t1, We2, s2, t2, Wmh0, bm0)],
        out_specs=(_row_spec(H), _agg_spec()),
        out_shape=(jax.ShapeDtypeStruct((N, H), jnp.float32),
                   jax.ShapeDtypeStruct((2, N, HH), jnp.float32)),
    )(x, We1, s1, t1, We2, s2, t2, Wmh0, bm0)


def _update_call(h, agg, Wuh, Wua, bu, Wmh_next, bm_next):
    def body(h_ref, agg_ref, wuh_ref, wua_ref, bu_ref, wm_ref, bm_ref,
             hn_ref, p_ref):
        h_in = h_ref[...]
        hn = (h_in + _dot(h_in, wuh_ref[...])
              + _dot(agg_ref[0], wua_ref[0])
              + _dot(agg_ref[1], wua_ref[1]) + bu_ref[...])
        hn_ref[...] = hn
        p = _dot(hn, wm_ref[...]) + bm_ref[...]
        p_ref[0] = p[:, :HH]
        p_ref[1] = p[:, HH:]

    return pl.pallas_call(
        body,
        grid=(_NB,),
        in_specs=[_row_spec(H), _agg_spec()] + [
            _full_spec(a.shape) for a in (Wuh, Wua, bu, Wmh_next, bm_next)],
        out_specs=(_row_spec(H), _agg_spec()),
        out_shape=(jax.ShapeDtypeStruct((N, H), jnp.float32),
                   jax.ShapeDtypeStruct((2, N, HH), jnp.float32)),
    )(h, agg, Wuh, Wua, bu, Wmh_next, bm_next)


def _final_call(h, agg, Wuh, Wua, bu, Wd1, bd1, Wd2, bd2):
    def body(h_ref, agg_ref, wuh_ref, wua_ref, bu_ref, wd1_ref, bd1_ref,
             wd2_ref, bd2_ref, out_ref):
        h_in = h_ref[...]
        hn = (h_in + _dot(h_in, wuh_ref[...])
              + _dot(agg_ref[0], wua_ref[0])
              + _dot(agg_ref[1], wua_ref[1]) + bu_ref[...])
        d1 = jnp.maximum(_dot(hn, wd1_ref[...]) + bd1_ref[...], 0.0)
        out_ref[...] = _dot(d1, wd2_ref[...]) + bd2_ref[...]

    return pl.pallas_call(
        body,
        grid=(_NB,),
        in_specs=[_row_spec(H), _agg_spec()] + [
            _full_spec(a.shape) for a in (Wuh, Wua, bu, Wd1, bd1, Wd2, bd2)],
        out_specs=_row_spec(3),
        out_shape=jax.ShapeDtypeStruct((N, 3), jnp.float32),
    )(h, agg, Wuh, Wua, bu, Wd1, bd1, Wd2, bd2)


def kernel(x, edge_index, edge_attr, We1, be1, g1, b1, m1, v1, We2, be2,
           g2, b2, m2, v2, Wm, bm, Wu, bu, Wd1, bd1, Wd2, bd2):
    L = Wm.shape[0]
    s1 = g1 / jnp.sqrt(v1 + 1e-5)
    t1 = (be1 - m1) * s1 + b1
    s2 = g2 / jnp.sqrt(v2 + 1e-5)
    t2 = (be2 - m2) * s2 + b2

    src = edge_index[0]
    srcoff = jnp.concatenate([src, src + N])
    dst1 = edge_index[1]
    attrc = (edge_attr.T.reshape(3, E // CE, CE).transpose(1, 0, 2)
             .reshape(3 * E).astype(jnp.bfloat16).astype(jnp.float32))

    Wmh = Wm[:, :H, :]
    Wme = Wm[:, H:, :]
    wme = (jnp.concatenate([Wme[:, :, :HH], Wme[:, :, HH:]], axis=1)
           .reshape(L, 6 * HH).astype(jnp.bfloat16).astype(jnp.float32))
    Wuh = Wu[:, :H, :]
    Wua = Wu[:, H:, :].reshape(L, 2, HH, H)

    h, p = _encoder_call(x, We1, s1, t1, We2, s2, t2, Wmh[0], bm[0])
    for l in range(L):
        aggflat = _edge_kernel(p.reshape(2 * N, HH), srcoff, dst1, attrc,
                               wme[l])
        agg = aggflat.reshape(2, N, HH)
        if l + 1 < L:
            h, p = _update_call(h, agg, Wuh[l], Wua[l], bu[l],
                                Wmh[l + 1], bm[l + 1])
        else:
            pred = _final_call(h, agg, Wuh[l], Wua[l], bu[l],
                               Wd1, bd1, Wd2, bd2)
    return pred


# speedup vs baseline: 9.3465x; 1.2912x over previous
import functools

import jax
import jax.numpy as jnp
from jax import lax
from jax.experimental import pallas as pl
from jax.experimental.pallas import tpu as pltpu
from jax.experimental.pallas import tpu_sc as plsc

N = 10000
E = 640000
H = 128
HH = 64
NSUB = 16
NCORE = 2
EPS = E // NSUB
G = 5
CE = G * 64
NCHUNK = EPS // CE
NPS = N // NSUB


def _edge_kernel(pflat, srcoff, dst1, attrc, wme1):
    mesh = plsc.VectorSubcoreMesh(core_axis_name="c", subcore_axis_name="s")

    @functools.partial(
        pl.kernel,
        out_type=jax.ShapeDtypeStruct((NCORE * N, HH), jnp.float32),
        mesh=mesh,
        compiler_params=pltpu.CompilerParams(use_tc_tiling_on_sc=False),
        scratch_types=(
            [pltpu.VMEM((64,), jnp.int32) for _ in range(3 * G)]
            + [pltpu.VMEM((CE,), jnp.int32) for _ in range(3)]
            + [pltpu.VMEM((3 * CE,), jnp.float32) for _ in range(3)]
            + [pltpu.VMEM((CE, HH), jnp.float32) for _ in range(3)]
            + [
                pltpu.VMEM((3 * HH,), jnp.float32),
                pltpu.VMEM_SHARED((N, HH), jnp.float32),
            ]
            + [pltpu.SemaphoreType.DMA for _ in range(12)]
        ),
    )
    def body(p_ref, src_ref, dst_ref, attr_ref, wme_ref, agg_ref, *scr):
        idxd = [scr[:G], scr[G:2 * G], scr[2 * G:3 * G]]
        o = 3 * G
        idxs = scr[o:o + 3]
        attr = scr[o + 3:o + 6]
        rows = scr[o + 6:o + 9]
        wm_v, agg_sh = scr[o + 9:o + 11]
        semi = scr[o + 11:o + 14]
        semid = scr[o + 14:o + 17]
        semg = scr[o + 17:o + 20]
        sems = scr[o + 20:o + 23]
        c = lax.axis_index("c")
        s = lax.axis_index("s")

        pltpu.sync_copy(wme_ref.at[pl.ds(c * 3 * HH, 3 * HH)], wm_v)

        zero = jnp.zeros((16,), jnp.float32)

        def zrow(i, carry):
            for k in range(HH // 16):
                rows[0][i, pl.ds(k * 16, 16)] = zero
            return carry

        lax.fori_loop(0, CE, zrow, 0)
        n0 = s * 624
        pltpu.sync_copy(rows[0], agg_sh.at[pl.ds(n0, CE)])
        pltpu.sync_copy(rows[0].at[pl.ds(0, 624 - CE)],
                        agg_sh.at[pl.ds(n0 + CE, 624 - CE)])

        @pl.when(s == 0)
        def _():
            pltpu.sync_copy(rows[0].at[pl.ds(0, 16)],
                            agg_sh.at[pl.ds(N - 16, 16)])

        plsc.subcore_barrier()

        w = [[wm_v[pl.ds(j * HH + k * 16, 16)] for k in range(HH // 16)]
             for j in range(3)]
        coff = c * N

        def fire_isa(b, cc):
            e0 = s * EPS + cc * CE
            pltpu.async_copy(src_ref.at[pl.ds(c * E + e0, CE)],
                             idxs[b], semi[b])
            pltpu.async_copy(attr_ref.at[pl.ds(3 * e0, 3 * CE)],
                             attr[b], semi[b])

        def wait_isa(b):
            pltpu.make_async_copy(src_ref.at[pl.ds(0, CE)],
                                  idxs[b], semi[b]).wait()
            pltpu.make_async_copy(attr_ref.at[pl.ds(0, 3 * CE)],
                                  attr[b], semi[b]).wait()

        def fire_id(b, cc):
            e0 = s * EPS + cc * CE
            for g in range(G):
                pltpu.async_copy(dst_ref.at[pl.ds(e0 + g * 64, 64)],
                                 idxd[b][g], semid[b])

        def wait_id(b):
            for g in range(G):
                pltpu.make_async_copy(dst_ref.at[pl.ds(0, 64)],
                                      idxd[b][g], semid[b]).wait()

        def fire_g(b):
            for g in range(G):
                pltpu.async_copy(p_ref.at[idxs[b].at[pl.ds(g * 64, 64)]],
                                 rows[b].at[pl.ds(g * 64, 64)], semg[b])

        def wait_g(b):
            pltpu.make_async_copy(p_ref.at[pl.ds(0, CE)],
                                  rows[b], semg[b]).wait()

        def fire_s(b):
            for g in range(G):
                pltpu.async_copy(rows[b].at[pl.ds(g * 64, 64)],
                                 agg_sh.at[idxd[b][g]], sems[b], add=True)

        def drain_s(b):
            pltpu.make_async_copy(p_ref.at[pl.ds(0, CE)],
                                  rows[b], sems[b]).wait()

        def compute(b):
            @plsc.parallel_loop(0, CE // 16, unroll=2)
            def grp(q):
                i0 = q * 16
                a0g = attr[b][pl.ds(i0, 16)]
                a1g = attr[b][pl.ds(CE + i0, 16)]
                a2g = attr[b][pl.ds(2 * CE + i0, 16)]
                for j in range(16):
                    i = i0 + j
                    a0 = a0g[j]
                    a1 = a1g[j]
                    a2 = a2g[j]
                    for k in range(HH // 16):
                        p = rows[b][i, pl.ds(k * 16, 16)]
                        m = jnp.maximum(
                            (p + a0 * w[0][k])
                            + (a1 * w[1][k] + a2 * w[2][k]), 0.0)
                        rows[b][i, pl.ds(k * 16, 16)] = m

        def position(b, cc, fire_next, fire_next2, guard0=False):
            b1 = (b + 1) % 3
            b2 = (b + 2) % 3
            wait_g(b)
            if fire_next:
                wait_isa(b1)
                fire_g(b1)
            wait_id(b)
            compute(b)
            fire_s(b)
            if guard0:
                @pl.when(cc > 0)
                def _():
                    drain_s(b2)
            else:
                drain_s(b2)
            if fire_next2:
                fire_isa(b2, cc + 2)
                fire_id(b2, cc + 2)

        fire_isa(0, 0)
        fire_id(0, 0)
        wait_isa(0)
        fire_g(0)
        fire_isa(1, 1)
        fire_id(1, 1)

        def pipe(t, carry):
            cc = 3 * t
            position(0, cc, True, True, guard0=True)
            position(1, cc + 1, True, True)
            position(2, cc + 2, True, True)
            return carry

        lax.fori_loop(0, (NCHUNK - 2) // 3, pipe, 0)
        position(0, NCHUNK - 2, True, False)
        position(1, NCHUNK - 1, False, False)
        drain_s(1)
        plsc.subcore_barrier()
        pltpu.sync_copy(agg_sh.at[pl.ds(n0, 624)],
                        agg_ref.at[pl.ds(coff + n0, 624)])

        @pl.when(s == 0)
        def _():
            pltpu.sync_copy(agg_sh.at[pl.ds(N - 16, 16)],
                            agg_ref.at[pl.ds(coff + N - 16, 16)])

    return body(pflat, srcoff, dst1, attrc, wme1)


def _dot(a, b):
    return jnp.dot(a, b, preferred_element_type=jnp.float32)


BR = 1000
_NB = N // BR


def _row_spec(cols):
    return pl.BlockSpec((BR, cols), lambda i: (i, 0))


def _agg_spec():
    return pl.BlockSpec((2, BR, HH), lambda i: (0, i, 0))


def _full_spec(shape):
    nd = len(shape)
    return pl.BlockSpec(shape, lambda i, _nd=nd: (0,) * nd)


def _encoder_call(x, We1, s1, t1, We2, s2, t2, Wmh0, bm0):
    def body(x_ref, we1_ref, s1_ref, t1_ref, we2_ref, s2_ref, t2_ref,
             wm_ref, bm_ref, h_ref, p_ref):
        h1 = jnp.maximum(_dot(x_ref[...], we1_ref[...]) * s1_ref[...]
                         + t1_ref[...], 0.0)
        h2 = jnp.maximum(_dot(h1, we2_ref[...]) * s2_ref[...]
                         + t2_ref[...], 0.0)
        h_ref[...] = h2
        p = _dot(h2, wm_ref[...]) + bm_ref[...]
        p_ref[0] = p[:, :HH]
        p_ref[1] = p[:, HH:]

    return pl.pallas_call(
        body,
        grid=(_NB,),
        in_specs=[_row_spec(H)] + [_full_spec(a.shape) for a in
                                   (We1, s1, t1, We2, s2, t2, Wmh0, bm0)],
        out_specs=(_row_spec(H), _agg_spec()),
        out_shape=(jax.ShapeDtypeStruct((N, H), jnp.float32),
                   jax.ShapeDtypeStruct((2, N, HH), jnp.float32)),
    )(x, We1, s1, t1, We2, s2, t2, Wmh0, bm0)


def _update_call(h, agg, Wuh, Wua, bu, Wmh_next, bm_next):
    def body(h_ref, agg_ref, wuh_ref, wua_ref, bu_ref, wm_ref, bm_ref,
             hn_ref, p_ref):
        h_in = h_ref[...]
        hn = (h_in + _dot(h_in, wuh_ref[...])
              + _dot(agg_ref[0], wua_ref[0])
              + _dot(agg_ref[1], wua_ref[1]) + bu_ref[...])
        hn_ref[...] = hn
        p = _dot(hn, wm_ref[...]) + bm_ref[...]
        p_ref[0] = p[:, :HH]
        p_ref[1] = p[:, HH:]

    return pl.pallas_call(
        body,
        grid=(_NB,),
        in_specs=[_row_spec(H), _agg_spec()] + [
            _full_spec(a.shape) for a in (Wuh, Wua, bu, Wmh_next, bm_next)],
        out_specs=(_row_spec(H), _agg_spec()),
        out_shape=(jax.ShapeDtypeStruct((N, H), jnp.float32),
                   jax.ShapeDtypeStruct((2, N, HH), jnp.float32)),
    )(h, agg, Wuh, Wua, bu, Wmh_next, bm_next)


def _final_call(h, agg, Wuh, Wua, bu, Wd1, bd1, Wd2, bd2):
    def body(h_ref, agg_ref, wuh_ref, wua_ref, bu_ref, wd1_ref, bd1_ref,
             wd2_ref, bd2_ref, out_ref):
        h_in = h_ref[...]
        hn = (h_in + _dot(h_in, wuh_ref[...])
              + _dot(agg_ref[0], wua_ref[0])
              + _dot(agg_ref[1], wua_ref[1]) + bu_ref[...])
        d1 = jnp.maximum(_dot(hn, wd1_ref[...]) + bd1_ref[...], 0.0)
        out_ref[...] = _dot(d1, wd2_ref[...]) + bd2_ref[...]

    return pl.pallas_call(
        body,
        grid=(_NB,),
        in_specs=[_row_spec(H), _agg_spec()] + [
            _full_spec(a.shape) for a in (Wuh, Wua, bu, Wd1, bd1, Wd2, bd2)],
        out_specs=_row_spec(3),
        out_shape=jax.ShapeDtypeStruct((N, 3), jnp.float32),
    )(h, agg, Wuh, Wua, bu, Wd1, bd1, Wd2, bd2)


def kernel(x, edge_index, edge_attr, We1, be1, g1, b1, m1, v1, We2, be2,
           g2, b2, m2, v2, Wm, bm, Wu, bu, Wd1, bd1, Wd2, bd2):
    L = Wm.shape[0]
    s1 = g1 / jnp.sqrt(v1 + 1e-5)
    t1 = (be1 - m1) * s1 + b1
    s2 = g2 / jnp.sqrt(v2 + 1e-5)
    t2 = (be2 - m2) * s2 + b2

    src = edge_index[0]
    srcoff = jnp.concatenate([src, src + N])
    dst1 = edge_index[1]
    attrc = (edge_attr.T.reshape(3, E // CE, CE).transpose(1, 0, 2)
             .reshape(3 * E).astype(jnp.bfloat16).astype(jnp.float32))

    Wmh = Wm[:, :H, :]
    Wme = Wm[:, H:, :]
    wme = (jnp.concatenate([Wme[:, :, :HH], Wme[:, :, HH:]], axis=1)
           .reshape(L, 6 * HH).astype(jnp.bfloat16).astype(jnp.float32))
    Wuh = Wu[:, :H, :]
    Wua = Wu[:, H:, :].reshape(L, 2, HH, H)

    h, p = _encoder_call(x, We1, s1, t1, We2, s2, t2, Wmh[0], bm[0])
    for l in range(L):
        aggflat = _edge_kernel(p.reshape(2 * N, HH), srcoff, dst1, attrc,
                               wme[l])
        agg = aggflat.reshape(2, N, HH)
        if l + 1 < L:
            h, p = _update_call(h, agg, Wuh[l], Wua[l], bu[l],
                                Wmh[l + 1], bm[l + 1])
        else:
            pred = _final_call(h, agg, Wuh[l], Wua[l], bu[l],
                               Wd1, bd1, Wd2, bd2)
    return pred
